# baseline (speedup 1.0000x reference)
import functools

import jax
import jax.numpy as jnp
from jax import lax
from jax.experimental import pallas as pl
from jax.experimental.pallas import tpu as pltpu
from jax.experimental.pallas import tpu_sc as plsc

B = 4096
F = 26
V = 1001
D = 29
DP = 32
G = 7
FP = 4 * G
CIN = G * 128

_NW = 32
_BPW = B // _NW
_TPW = _BPW // 8
_NPW = _BPW * FP


def _gather_body(table_hbm, idx_hbm, out_hbm, idx_v, rows_v, sem):
    c = lax.axis_index("c")
    s = lax.axis_index("s")
    wid = s * 2 + c
    p0 = wid * _NPW

    pltpu.sync_copy(idx_hbm.at[pl.ds(p0, _NPW)], idx_v)
    fi_off = (lax.iota(jnp.int32, 16) & 3) * V
    for i in range(_NPW // 16):
        g = (i * 16 // 32) % G
        sl = pl.ds(i * 16, 16)
        idx_v[sl] = idx_v[sl] + (4 * g * V) + fi_off
    pltpu.async_copy(table_hbm.at[idx_v], rows_v, sem).wait()
    zero = jnp.zeros((16,), jnp.float32)
    for t in range(_TPW):
        for sub in range(8):
            r = ((t * G + 6) * 8 + sub) * 4 + 2
            for q in range(2):
                rows_v[r, pl.ds(q * 16, 16)] = zero
                rows_v[r + 1, pl.ds(q * 16, 16)] = zero
    pltpu.sync_copy(rows_v, out_hbm.at[pl.ds(p0, _NPW), :])


def _sc_gather(table_flat, idx_perm):
    mesh = plsc.VectorSubcoreMesh(core_axis_name="c", subcore_axis_name="s")
    k = functools.partial(
        pl.kernel,
        mesh=mesh,
        out_type=jax.ShapeDtypeStruct((B * FP, DP), jnp.float32),
        scratch_types=[
            pltpu.VMEM((_NPW,), jnp.int32),
            pltpu.VMEM((_NPW, DP), jnp.float32),
            pltpu.SemaphoreType.DMA,
        ],
        compiler_params=pltpu.CompilerParams(use_tc_tiling_on_sc=False),
    )(_gather_body)
    return k(table_flat, idx_perm)


def _mlp_body(x_ref, w1, b1, w2, b2, w3, b3, w4, b4, w5, b5, w6, b6, o_ref):
    v = x_ref[...]
    v4 = v.reshape(_BB // 8, G, 8, 128)
    h = b1[...]
    for g in range(G):
        xg = v4[:, g].reshape(_BB, 128)
        h = h + jnp.dot(xg, w1[pl.ds(g * 128, 128), :], preferred_element_type=jnp.float32)
    h = jnp.maximum(h, 0.0)
    h = jnp.maximum(jnp.dot(h, w2[...], preferred_element_type=jnp.float32) + b2[...], 0.0)
    h = jnp.maximum(jnp.dot(h, w3[...], preferred_element_type=jnp.float32) + b3[...], 0.0)
    h = jnp.maximum(jnp.dot(h, w4[...], preferred_element_type=jnp.float32) + b4[...], 0.0)
    h = jnp.maximum(jnp.dot(h, w5[...], preferred_element_type=jnp.float32) + b5[...], 0.0)
    z = jnp.dot(h, w6[...], preferred_element_type=jnp.float32) + b6[...]
    o_ref[...] = jax.nn.sigmoid(z)


_BB = 512


def _tc_mlp(x128, w1, b1, w2, b2, w3, b3, w4, b4, w5, b5, w6, b6):
    full = lambda a: pl.BlockSpec(a.shape, lambda i: (0, 0))
    return pl.pallas_call(
        _mlp_body,
        grid=(B // _BB,),
        in_specs=[pl.BlockSpec((_BB * G, 128), lambda i: (i, 0))]
        + [full(a) for a in (w1, b1, w2, b2, w3, b3, w4, b4, w5, b5, w6, b6)],
        out_specs=pl.BlockSpec((_BB, 1), lambda i: (i, 0)),
        out_shape=jax.ShapeDtypeStruct((B, 1), jnp.float32),
    )(x128, w1, b1, w2, b2, w3, b3, w4, b4, w5, b5, w6, b6)


def kernel(indices, emb_tables, W1, b1, W2, b2, W3, b3, W4, b4, W5, b5, W6, b6):
    table_flat = jnp.pad(emb_tables, ((0, 0), (0, 0), (0, DP - D))).reshape(F * V, DP)
    dummy = jnp.broadcast_to(
        jnp.array([-(F * V), -((F + 1) * V)], dtype=jnp.int32), (B, 2)
    )
    idx_ext = jnp.concatenate([indices.astype(jnp.int32), dummy], axis=1)
    idx_perm = (
        idx_ext.reshape(_NW, _TPW, 8, G, 4).transpose(0, 1, 3, 2, 4).reshape(-1)
    )
    w1p = jnp.pad(W1.reshape(F, D, -1), ((0, FP - F), (0, DP - D), (0, 0))).reshape(CIN, -1)

    x = _sc_gather(table_flat, idx_perm)
    x128 = x.reshape(B * G, 128)

    args = (w1p, b1, W2, b2, W3, b3, W4, b4, W5, b5, W6, b6)
    args = tuple(a if a.ndim == 2 else a.reshape(1, -1) for a in args)
    return _tc_mlp(x128, *args)


# speedup vs baseline: 1.0017x; 1.0017x over previous
import functools

import jax
import jax.numpy as jnp
from jax import lax
from jax.experimental import pallas as pl
from jax.experimental.pallas import tpu as pltpu
from jax.experimental.pallas import tpu_sc as plsc

B = 4096
F = 26
V = 1001
D = 29
DP = 32
G = 7
FP = 4 * G
CIN = G * 128

_NW = 32
_BPW = B // _NW
_TPW = _BPW // 8
_NPW = _BPW * FP


def _gather_body(table_hbm, idx_hbm, out_hbm, idx_v, rows_v, sem):
    c = lax.axis_index("c")
    s = lax.axis_index("s")
    wid = s * 2 + c
    p0 = wid * _NPW

    pltpu.sync_copy(idx_hbm.at[pl.ds(p0, _NPW)], idx_v)
    fi_off = (lax.iota(jnp.int32, 16) & 3) * V
    for i in range(_NPW // 16):
        g = (i * 16 // 32) % G
        sl = pl.ds(i * 16, 16)
        idx_v[sl] = idx_v[sl] + (4 * g * V) + fi_off
    pltpu.async_copy(table_hbm.at[idx_v], rows_v, sem).wait()
    pltpu.sync_copy(rows_v, out_hbm.at[pl.ds(p0, _NPW), :])


def _sc_gather(table_flat, idx_perm):
    mesh = plsc.VectorSubcoreMesh(core_axis_name="c", subcore_axis_name="s")
    k = functools.partial(
        pl.kernel,
        mesh=mesh,
        out_type=jax.ShapeDtypeStruct((B * FP, DP), jnp.float32),
        scratch_types=[
            pltpu.VMEM((_NPW,), jnp.int32),
            pltpu.VMEM((_NPW, DP), jnp.float32),
            pltpu.SemaphoreType.DMA,
        ],
        compiler_params=pltpu.CompilerParams(use_tc_tiling_on_sc=False),
    )(_gather_body)
    return k(table_flat, idx_perm)


def _mlp_body(x_ref, w1, b1, w2, b2, w3, b3, w4, b4, w5, b5, w6, b6, o_ref):
    v = x_ref[...]
    v4 = v.reshape(_BB // 8, G, 8, 128)
    h = b1[...]
    for g in range(G):
        xg = v4[:, g].reshape(_BB, 128)
        h = h + jnp.dot(xg, w1[pl.ds(g * 128, 128), :], preferred_element_type=jnp.float32)
    h = jnp.maximum(h, 0.0)
    h = jnp.maximum(jnp.dot(h, w2[...], preferred_element_type=jnp.float32) + b2[...], 0.0)
    h = jnp.maximum(jnp.dot(h, w3[...], preferred_element_type=jnp.float32) + b3[...], 0.0)
    h = jnp.maximum(jnp.dot(h, w4[...], preferred_element_type=jnp.float32) + b4[...], 0.0)
    h = jnp.maximum(jnp.dot(h, w5[...], preferred_element_type=jnp.float32) + b5[...], 0.0)
    z = jnp.dot(h, w6[...], preferred_element_type=jnp.float32) + b6[...]
    o_ref[...] = jax.nn.sigmoid(z)


_BB = 512


def _tc_mlp(x128, w1, b1, w2, b2, w3, b3, w4, b4, w5, b5, w6, b6):
    full = lambda a: pl.BlockSpec(a.shape, lambda i: (0, 0))
    return pl.pallas_call(
        _mlp_body,
        grid=(B // _BB,),
        in_specs=[pl.BlockSpec((_BB * G, 128), lambda i: (i, 0))]
        + [full(a) for a in (w1, b1, w2, b2, w3, b3, w4, b4, w5, b5, w6, b6)],
        out_specs=pl.BlockSpec((_BB, 1), lambda i: (i, 0)),
        out_shape=jax.ShapeDtypeStruct((B, 1), jnp.float32),
    )(x128, w1, b1, w2, b2, w3, b3, w4, b4, w5, b5, w6, b6)


def kernel(indices, emb_tables, W1, b1, W2, b2, W3, b3, W4, b4, W5, b5, W6, b6):
    table_flat = jnp.pad(emb_tables, ((0, 0), (0, 0), (0, DP - D))).reshape(F * V, DP)
    dummy = jnp.broadcast_to(
        jnp.array([-(F * V), -((F + 1) * V)], dtype=jnp.int32), (B, 2)
    )
    idx_ext = jnp.concatenate([indices.astype(jnp.int32), dummy], axis=1)
    idx_perm = (
        idx_ext.reshape(_NW, _TPW, 8, G, 4).transpose(0, 1, 3, 2, 4).reshape(-1)
    )
    w1p = jnp.pad(W1.reshape(F, D, -1), ((0, FP - F), (0, DP - D), (0, 0))).reshape(CIN, -1)

    x = _sc_gather(table_flat, idx_perm)
    x128 = x.reshape(B * G, 128)

    args = (w1p, b1, W2, b2, W3, b3, W4, b4, W5, b5, W6, b6)
    args = tuple(a if a.ndim == 2 else a.reshape(1, -1) for a in args)
    return _tc_mlp(x128, *args)


# speedup vs baseline: 1.7591x; 1.7562x over previous
import functools

import jax
import jax.numpy as jnp
from jax import lax
from jax.experimental import pallas as pl
from jax.experimental.pallas import tpu as pltpu
from jax.experimental.pallas import tpu_sc as plsc

B = 4096
F = 26
V = 1001
D = 29
DP = 32
G = 7
FP = 4 * G
CIN = G * 128

_NW = 32
_BPW = B // _NW
_TPW = _BPW // 8
_NPW = _BPW * FP


def _gather_body(table_hbm, idx_hbm, out_hbm, idx_v, rows_v, sem):
    c = lax.axis_index("c")
    s = lax.axis_index("s")
    wid = s * 2 + c
    p0 = wid * _NPW

    pltpu.sync_copy(idx_hbm.at[pl.ds(p0, _NPW)], idx_v)
    fi_off = (lax.iota(jnp.int32, 16) & 3) * V
    for i in range(_NPW // 16):
        g = (i * 16 // 32) % G
        sl = pl.ds(i * 16, 16)
        idx_v[sl] = idx_v[sl] + (4 * g * V) + fi_off
    pltpu.async_copy(table_hbm.at[idx_v], rows_v, sem).wait()
    pltpu.sync_copy(rows_v, out_hbm.at[pl.ds(p0, _NPW), :])


def _sc_gather(table_flat, idx_perm):
    mesh = plsc.VectorSubcoreMesh(core_axis_name="c", subcore_axis_name="s")
    k = functools.partial(
        pl.kernel,
        mesh=mesh,
        out_type=jax.ShapeDtypeStruct((B * FP, DP), jnp.float32),
        scratch_types=[
            pltpu.VMEM((_NPW,), jnp.int32),
            pltpu.VMEM((_NPW, DP), jnp.float32),
            pltpu.SemaphoreType.DMA,
        ],
        compiler_params=pltpu.CompilerParams(use_tc_tiling_on_sc=False),
    )(_gather_body)
    return k(table_flat, idx_perm)


def _mlp_body(x_ref, w1, b1, w2, b2, w3, b3, w4, b4, w5, b5, w6, b6, o_ref):
    v = x_ref[...]
    v4 = v.reshape(_BB // 8, G, 8, 128)
    h = b1[...]
    for g in range(G):
        xg = v4[:, g].reshape(_BB, 128)
        h = h + jnp.dot(xg, w1[pl.ds(g * 128, 128), :], preferred_element_type=jnp.float32)
    h = jnp.maximum(h, 0.0)
    h = jnp.maximum(jnp.dot(h, w2[...], preferred_element_type=jnp.float32) + b2[...], 0.0)
    h = jnp.maximum(jnp.dot(h, w3[...], preferred_element_type=jnp.float32) + b3[...], 0.0)
    h = jnp.maximum(jnp.dot(h, w4[...], preferred_element_type=jnp.float32) + b4[...], 0.0)
    h = jnp.maximum(jnp.dot(h, w5[...], preferred_element_type=jnp.float32) + b5[...], 0.0)
    z = jnp.dot(h, w6[...], preferred_element_type=jnp.float32) + b6[...]
    o_ref[...] = jax.nn.sigmoid(z)


_BB = 512


def _tc_mlp(x128, w1, b1, w2, b2, w3, b3, w4, b4, w5, b5, w6, b6):
    full = lambda a: pl.BlockSpec(a.shape, lambda i: (0, 0))
    return pl.pallas_call(
        _mlp_body,
        grid=(B // _BB,),
        in_specs=[pl.BlockSpec((_BB * G, 128), lambda i: (i, 0))]
        + [full(a) for a in (w1, b1, w2, b2, w3, b3, w4, b4, w5, b5, w6, b6)],
        out_specs=pl.BlockSpec((_BB, 1), lambda i: (i, 0)),
        out_shape=jax.ShapeDtypeStruct((B, 1), jnp.float32),
    )(x128, w1, b1, w2, b2, w3, b3, w4, b4, w5, b5, w6, b6)


def kernel(indices, emb_tables, W1, b1, W2, b2, W3, b3, W4, b4, W5, b5, W6, b6):
    table_flat = jnp.pad(emb_tables, ((0, 0), (0, 0), (0, DP - D))).reshape(F * V, DP)
    spread = jnp.arange(B, dtype=jnp.int32) % (F * V)
    dummy = jnp.stack([spread - F * V, spread - (F + 1) * V], axis=1)
    idx_ext = jnp.concatenate([indices.astype(jnp.int32), dummy], axis=1)
    idx_perm = (
        idx_ext.reshape(_NW, _TPW, 8, G, 4).transpose(0, 1, 3, 2, 4).reshape(-1)
    )
    w1p = jnp.pad(W1.reshape(F, D, -1), ((0, FP - F), (0, DP - D), (0, 0))).reshape(CIN, -1)

    x = _sc_gather(table_flat, idx_perm)
    x128 = x.reshape(B * G, 128)

    args = (w1p, b1, W2, b2, W3, b3, W4, b4, W5, b5, W6, b6)
    args = tuple(a if a.ndim == 2 else a.reshape(1, -1) for a in args)
    return _tc_mlp(x128, *args)


# speedup vs baseline: 2.4990x; 1.4206x over previous
import functools

import jax
import jax.numpy as jnp
import numpy as np
from jax import lax
from jax.experimental import pallas as pl
from jax.experimental.pallas import tpu as pltpu
from jax.experimental.pallas import tpu_sc as plsc

B = 4096
F = 26
V = 1001
D = 29
DP = 32
G = 7
FP = 4 * G
CIN = G * 128

_NW = 32
_BPW = B // _NW
_TPW = _BPW // 8
_NPW = _BPW * FP


_NRW = _BPW * F


def _perm_offs():
    p = np.arange(_NPW)
    t, r = p // 224, p % 224
    g, q = r // 32, r % 32
    sub, fi = q // 4, q % 4
    f = 4 * g + fi
    b_local = 8 * t + sub
    valid = f < F
    perm = np.where(valid, b_local * F + np.minimum(f, F - 1), 0)
    offs = np.where(valid, f * V, (p * 997) % (F * V - V))
    return (jnp.asarray(perm, dtype=jnp.int32), jnp.asarray(offs, dtype=jnp.int32))


def _gather_body(table_hbm, idx_hbm, perm_hbm, offs_hbm, out_hbm,
                 raw_v, perm_v, offs_v, rows_v, sem):
    c = lax.axis_index("c")
    s = lax.axis_index("s")
    wid = s * 2 + c

    pltpu.sync_copy(idx_hbm.at[pl.ds(wid * _NRW, _NRW)], raw_v)
    pltpu.sync_copy(perm_hbm, perm_v)
    pltpu.sync_copy(offs_hbm, offs_v)
    for i in range(_NPW // 16):
        sl = pl.ds(i * 16, 16)
        pv = perm_v[sl]
        gathered = plsc.load_gather(raw_v, [pv])
        perm_v[sl] = gathered + offs_v[sl]
    pltpu.async_copy(table_hbm.at[perm_v], rows_v, sem).wait()
    pltpu.sync_copy(rows_v, out_hbm.at[pl.ds(wid * _NPW, _NPW), :])


def _sc_gather(table_flat, idx_raw, perm, offs):
    mesh = plsc.VectorSubcoreMesh(core_axis_name="c", subcore_axis_name="s")
    k = functools.partial(
        pl.kernel,
        mesh=mesh,
        out_type=jax.ShapeDtypeStruct((B * FP, DP), jnp.float32),
        scratch_types=[
            pltpu.VMEM((_NRW,), jnp.int32),
            pltpu.VMEM((_NPW,), jnp.int32),
            pltpu.VMEM((_NPW,), jnp.int32),
            pltpu.VMEM((_NPW, DP), jnp.float32),
            pltpu.SemaphoreType.DMA,
        ],
        compiler_params=pltpu.CompilerParams(
            use_tc_tiling_on_sc=False, needs_layout_passes=False
        ),
    )(_gather_body)
    return k(table_flat, idx_raw, perm, offs)


def _mlp_body(x_ref, w1, b1, w2, b2, w3, b3, w4, b4, w5, b5, w6, b6, o_ref):
    v = x_ref[...]
    v4 = v.reshape(_BB // 8, G, 8, 128)
    h = b1[...]
    for g in range(G):
        xg = v4[:, g].reshape(_BB, 128)
        h = h + jnp.dot(xg, w1[pl.ds(g * 128, 128), :], preferred_element_type=jnp.float32)
    h = jnp.maximum(h, 0.0)
    h = jnp.maximum(jnp.dot(h, w2[...], preferred_element_type=jnp.float32) + b2[...], 0.0)
    h = jnp.maximum(jnp.dot(h, w3[...], preferred_element_type=jnp.float32) + b3[...], 0.0)
    h = jnp.maximum(jnp.dot(h, w4[...], preferred_element_type=jnp.float32) + b4[...], 0.0)
    h = jnp.maximum(jnp.dot(h, w5[...], preferred_element_type=jnp.float32) + b5[...], 0.0)
    z = jnp.dot(h, w6[...], preferred_element_type=jnp.float32) + b6[...]
    o_ref[...] = jax.nn.sigmoid(z)


_BB = 512


def _tc_mlp(x128, w1, b1, w2, b2, w3, b3, w4, b4, w5, b5, w6, b6):
    full = lambda a: pl.BlockSpec(a.shape, lambda i: (0, 0))
    return pl.pallas_call(
        _mlp_body,
        grid=(B // _BB,),
        in_specs=[pl.BlockSpec((_BB * G, 128), lambda i: (i, 0))]
        + [full(a) for a in (w1, b1, w2, b2, w3, b3, w4, b4, w5, b5, w6, b6)],
        out_specs=pl.BlockSpec((_BB, 1), lambda i: (i, 0)),
        out_shape=jax.ShapeDtypeStruct((B, 1), jnp.float32),
    )(x128, w1, b1, w2, b2, w3, b3, w4, b4, w5, b5, w6, b6)


def kernel(indices, emb_tables, W1, b1, W2, b2, W3, b3, W4, b4, W5, b5, W6, b6):
    table_flat = jnp.pad(emb_tables, ((0, 0), (0, 0), (0, DP - D))).reshape(F * V, DP)
    idx_raw = indices.astype(jnp.int32).reshape(B * F)
    perm, offs = _perm_offs()
    w1p = jnp.pad(W1.reshape(F, D, -1), ((0, FP - F), (0, DP - D), (0, 0))).reshape(CIN, -1)

    x = _sc_gather(table_flat, idx_raw, perm, offs)
    x128 = x.reshape(B * G, 128)

    args = (w1p, b1, W2, b2, W3, b3, W4, b4, W5, b5, W6, b6)
    args = tuple(a if a.ndim == 2 else a.reshape(1, -1) for a in args)
    return _tc_mlp(x128, *args)
